# Initial kernel scaffold; baseline (speedup 1.0000x reference)
import jax
import jax.numpy as jnp
from jax.experimental import pallas as pl


def kernel(entity_ids, edge_index, entity_table, W1, a_src1, a_dst1, b1, g1, be1, W2, a_src2, a_dst2, b2, g2, be2):
    raise NotImplementedError("write your pallas kernel here")


# speedup vs baseline: 3.7099x; 3.7099x over previous
import functools

import jax
import jax.numpy as jnp
from jax import lax
from jax.experimental import pallas as pl
from jax.experimental.pallas import tpu as pltpu
from jax.experimental.pallas import tpu_sc as plsc

N_NODES = 10000
N_EDGES = 160000
D = 256
HID = 256
HEADS = 8

NC = 2
NS = 16
NW = NC * NS
L = 16

NPAD = 10240
EPAD = 174080
KB = 64
NB_W = EPAD // NW // KB
ROWS_W = NPAD // NW
STRIPE = NPAD // NS

_MESH = plsc.VectorSubcoreMesh(core_axis_name="c", subcore_axis_name="s")
_SC_PARAMS = pltpu.CompilerParams(use_tc_tiling_on_sc=False,
                                  needs_layout_passes=False)


def _gather_body(table_hbm, idx_hbm, out_hbm, idx_v, rows_v, sem):
    cid = lax.axis_index("c")
    sid = lax.axis_index("s")
    wid = sid * NC + cid
    pltpu.sync_copy(idx_hbm.at[wid], idx_v)
    descs = []
    for j in range(ROWS_W // KB):
        descs.append(
            pltpu.async_copy(table_hbm.at[idx_v.at[j]],
                             rows_v.at[pl.ds(j * KB, KB)], sem))
    for d in descs:
        d.wait()
    pltpu.sync_copy(rows_v, out_hbm.at[pl.ds(wid * ROWS_W, ROWS_W)])


_gather_rows = pl.kernel(
    _gather_body,
    out_type=jax.ShapeDtypeStruct((NPAD, D), jnp.float32),
    mesh=_MESH,
    compiler_params=_SC_PARAMS,
    scratch_types=[
        pltpu.VMEM((ROWS_W // KB, KB), jnp.int32),
        pltpu.VMEM((ROWS_W, D), jnp.float32),
        pltpu.SemaphoreType.DMA,
    ],
)


def _edge_w_body(ta_hbm, tb_hbm, srcw_hbm, dstw_hbm, w_hbm, den_hbm,
                 idx_s, idx_d, buf_a, buf_b, buf_w, den_sh, sem):
    cid = lax.axis_index("c")
    sid = lax.axis_index("s")
    wid = sid * NC + cid
    pltpu.sync_copy(srcw_hbm.at[wid], idx_s)
    pltpu.sync_copy(dstw_hbm.at[wid], idx_d)

    def zr(r, _):
        buf_w[r, :] = jnp.zeros((L,), jnp.float32)
        return 0
    lax.fori_loop(0, KB, zr, 0)
    for t in range(STRIPE // KB):
        pltpu.sync_copy(buf_w, den_sh.at[pl.ds(sid * STRIPE + t * KB, KB)])
    plsc.subcore_barrier()

    def jb(j, _):
        pltpu.async_copy(ta_hbm.at[idx_s.at[j]], buf_a, sem).wait()
        pltpu.async_copy(tb_hbm.at[idx_d.at[j]], buf_b, sem).wait()

        def ek(k, _):
            e = buf_a[k, :] + buf_b[k, :]
            e = jnp.where(e >= 0.0, e, 0.2 * e)
            buf_w[k, :] = jnp.exp(e)
            return 0
        lax.fori_loop(0, KB, ek, 0)
        pltpu.sync_copy(buf_w, den_sh.at[idx_d.at[j]], add=True)
        pltpu.sync_copy(buf_w, w_hbm.at[wid].at[j])
        return 0
    lax.fori_loop(0, NB_W, jb, 0)
    plsc.subcore_barrier()
    pltpu.sync_copy(den_sh.at[pl.ds(sid * STRIPE, STRIPE)],
                    den_hbm.at[cid].at[pl.ds(sid * STRIPE, STRIPE)])


_edge_w = pl.kernel(
    _edge_w_body,
    out_type=(
        jax.ShapeDtypeStruct((NW, NB_W, KB, L), jnp.float32),
        jax.ShapeDtypeStruct((NC, NPAD, L), jnp.float32),
    ),
    mesh=_MESH,
    compiler_params=_SC_PARAMS,
    scratch_types=[
        pltpu.VMEM((NB_W, KB), jnp.int32),
        pltpu.VMEM((NB_W, KB), jnp.int32),
        pltpu.VMEM((KB, L), jnp.float32),
        pltpu.VMEM((KB, L), jnp.float32),
        pltpu.VMEM((KB, L), jnp.float32),
        pltpu.VMEM_SHARED((NPAD, L), jnp.float32),
        pltpu.SemaphoreType.DMA,
    ],
)


def _make_msg(nch, head_lanes):
    dtot = nch * 128

    def body(*refs):
        tabs = refs[:nch]
        srcw_hbm, dstw_hbm, w_hbm, out_hbm = refs[nch:nch + 4]
        idx_s, idx_d, rbuf, wbuf, zbuf, acc_sh, sem = refs[nch + 4:]
        cid = lax.axis_index("c")
        sid = lax.axis_index("s")
        wid = sid * NC + cid
        pltpu.sync_copy(srcw_hbm.at[wid], idx_s)
        pltpu.sync_copy(dstw_hbm.at[wid], idx_d)

        def zr(r, _):
            for t in range(8):
                zbuf[r, pl.ds(t * L, L)] = jnp.zeros((L,), jnp.float32)
            return 0
        lax.fori_loop(0, KB, zr, 0)

        for ci in range(nch):
            hl = head_lanes[ci]
            for t in range(STRIPE // KB):
                pltpu.sync_copy(zbuf, acc_sh.at[pl.ds(sid * STRIPE + t * KB, KB)])
            plsc.subcore_barrier()

            def jb(j, _):
                pltpu.async_copy(tabs[ci].at[idx_s.at[j]], rbuf, sem).wait()
                pltpu.sync_copy(w_hbm.at[wid].at[j], wbuf)

                def kb_(k, _):
                    wv = plsc.load_gather(
                        wbuf,
                        [jnp.full((L,), k, jnp.int32),
                         jnp.full((L,), hl, jnp.int32)])
                    for t in range(8):
                        rbuf[k, pl.ds(t * L, L)] = rbuf[k, pl.ds(t * L, L)] * wv
                    return 0
                lax.fori_loop(0, KB, kb_, 0)
                pltpu.sync_copy(rbuf, acc_sh.at[idx_d.at[j]], add=True)
                return 0
            lax.fori_loop(0, NB_W, jb, 0)
            plsc.subcore_barrier()
            pltpu.sync_copy(
                acc_sh.at[pl.ds(sid * STRIPE, STRIPE)],
                out_hbm.at[cid].at[pl.ds(sid * STRIPE, STRIPE),
                                   pl.ds(ci * 128, 128)])
            plsc.subcore_barrier()

    return pl.kernel(
        body,
        out_type=jax.ShapeDtypeStruct((NC, NPAD, dtot), jnp.float32),
        mesh=_MESH,
        compiler_params=_SC_PARAMS,
        scratch_types=[
            pltpu.VMEM((NB_W, KB), jnp.int32),
            pltpu.VMEM((NB_W, KB), jnp.int32),
            pltpu.VMEM((KB, 128), jnp.float32),
            pltpu.VMEM((KB, L), jnp.float32),
            pltpu.VMEM((KB, 128), jnp.float32),
            pltpu.VMEM_SHARED((NPAD, 128), jnp.float32),
            pltpu.SemaphoreType.DMA,
        ],
    )


_msg16 = _make_msg(16, [c // 2 for c in range(16)])
_msg2 = _make_msg(2, [0, 0])


def _mm1_body(x_ref, w_ref, asd_ref, xw_ref, la_ref):
    c = pl.program_id(1)
    acc = jnp.dot(x_ref[...], w_ref[...], preferred_element_type=jnp.float32)
    xw_ref[...] = acc.reshape(1, 256, 128)
    la = jnp.dot(acc, asd_ref[...], preferred_element_type=jnp.float32)

    @pl.when(c == 0)
    def _():
        la_ref[...] = la

    @pl.when(c != 0)
    def _():
        la_ref[...] = la_ref[...] + la


def _mm1(x, w1, asd1):
    return pl.pallas_call(
        _mm1_body,
        grid=(NPAD // 256, 16),
        in_specs=[
            pl.BlockSpec((256, D), lambda i, c: (i, 0)),
            pl.BlockSpec((D, 128), lambda i, c: (0, c)),
            pl.BlockSpec((128, 128), lambda i, c: (c, 0)),
        ],
        out_specs=[
            pl.BlockSpec((1, 256, 128), lambda i, c: (c, i, 0)),
            pl.BlockSpec((256, 128), lambda i, c: (i, 0)),
        ],
        out_shape=[
            jax.ShapeDtypeStruct((16, NPAD, 128), jnp.float32),
            jax.ShapeDtypeStruct((NPAD, 128), jnp.float32),
        ],
    )(x, w1, asd1)


def _mm2_body(h_ref, w_ref, asd_ref, xw_ref, la_ref):
    c = pl.program_id(1)
    acc = jnp.dot(h_ref[...], w_ref[...], preferred_element_type=jnp.float32)
    xw_ref[...] = acc.reshape(1, 256, 128)
    la = jnp.dot(acc, asd_ref[...], preferred_element_type=jnp.float32)

    @pl.when(c == 0)
    def _():
        la_ref[...] = la

    @pl.when(c != 0)
    def _():
        la_ref[...] = la_ref[...] + la


def _mm2(h, w2, asd2):
    return pl.pallas_call(
        _mm2_body,
        grid=(NPAD // 256, 2),
        in_specs=[
            pl.BlockSpec((256, HEADS * HID), lambda i, c: (i, 0)),
            pl.BlockSpec((HEADS * HID, 128), lambda i, c: (0, c)),
            pl.BlockSpec((128, 128), lambda i, c: (c, 0)),
        ],
        out_specs=[
            pl.BlockSpec((1, 256, 128), lambda i, c: (c, i, 0)),
            pl.BlockSpec((256, 128), lambda i, c: (i, 0)),
        ],
        out_shape=[
            jax.ShapeDtypeStruct((2, NPAD, 128), jnp.float32),
            jax.ShapeDtypeStruct((NPAD, 128), jnp.float32),
        ],
    )(h, w2, asd2)


def _epi_body(num_ref, den_ref, m_ref, b_ref, g_ref, be_ref, out_ref):
    den = den_ref[0] + den_ref[1]
    dexp = jnp.dot(den, m_ref[...], preferred_element_type=jnp.float32)
    num = num_ref[0] + num_ref[1]
    h = num / (dexp + 1e-16) + b_ref[...]
    mu = jnp.mean(h, axis=-1, keepdims=True)
    var = jnp.mean((h - mu) ** 2, axis=-1, keepdims=True)
    hn = (h - mu) / jnp.sqrt(var + 1e-5) * g_ref[...] + be_ref[...]
    out_ref[...] = jnp.where(hn > 0.0, hn, jnp.exp(hn) - 1.0)


def _make_epi(dt):
    def run(num, den, m, b, g, be):
        return pl.pallas_call(
            _epi_body,
            grid=(NPAD // 256,),
            in_specs=[
                pl.BlockSpec((2, 256, dt), lambda i: (0, i, 0)),
                pl.BlockSpec((2, 256, L), lambda i: (0, i, 0)),
                pl.BlockSpec((L, dt), lambda i: (0, 0)),
                pl.BlockSpec((1, dt), lambda i: (0, 0)),
                pl.BlockSpec((1, dt), lambda i: (0, 0)),
                pl.BlockSpec((1, dt), lambda i: (0, 0)),
            ],
            out_specs=pl.BlockSpec((256, dt), lambda i: (i, 0)),
            out_shape=jax.ShapeDtypeStruct((NPAD, dt), jnp.float32),
        )(num, den, m, b, g, be)
    return run


_epi1 = _make_epi(HEADS * HID)
_epi2 = _make_epi(D)


def kernel(entity_ids, edge_index, entity_table,
           W1, a_src1, a_dst1, b1, g1, be1,
           W2, a_src2, a_dst2, b2, g2, be2):
    f32 = jnp.float32
    ids = entity_ids.astype(jnp.int32)
    pad_ids = jnp.concatenate(
        [ids, jnp.zeros((NPAD - N_NODES,), jnp.int32)]).reshape(
            NW, ROWS_W // KB, KB)
    loop = jnp.arange(N_NODES, dtype=jnp.int32)
    padc = jnp.full((EPAD - N_EDGES - N_NODES,), NPAD - 1, jnp.int32)
    src = jnp.concatenate([edge_index[0].astype(jnp.int32), loop, padc])
    dst = jnp.concatenate([edge_index[1].astype(jnp.int32), loop, padc])
    srcw = src.reshape(NW, NB_W, KB)
    dstw = dst.reshape(NW, NB_W, KB)

    eye8 = jnp.eye(HEADS, dtype=f32)
    as_mat1 = (eye8[:, None, :] * a_src1[:, :, None]).reshape(HEADS * HID, HEADS)
    ad_mat1 = (eye8[:, None, :] * a_dst1[:, :, None]).reshape(HEADS * HID, HEADS)
    z8 = jnp.zeros((HEADS * HID, 8), f32)
    asd1 = jnp.concatenate(
        [as_mat1, z8, ad_mat1, jnp.zeros((HEADS * HID, 128 - 24), f32)], axis=1)
    asd2 = jnp.concatenate(
        [a_src2.T, jnp.zeros((D, 15), f32),
         a_dst2.T, jnp.zeros((D, 128 - 17), f32)], axis=1)
    m1 = jnp.concatenate(
        [jnp.kron(eye8, jnp.ones((1, HID), f32)),
         jnp.zeros((8, HEADS * HID), f32)], axis=0)
    m2 = jnp.concatenate([jnp.ones((1, D), f32), jnp.zeros((15, D), f32)], axis=0)

    x = _gather_rows(entity_table, pad_ids)
    xw_ch, asad1 = _mm1(x, W1, asd1)
    w1e, den1 = _edge_w(asad1[:, 0:16], asad1[:, 16:32], srcw, dstw)
    num1 = _msg16(*[xw_ch[c] for c in range(16)], srcw, dstw, w1e)
    h1 = _epi1(num1, den1, m1, b1.reshape(1, -1), g1.reshape(1, -1),
               be1.reshape(1, -1))
    xw2_ch, asad2_l = _mm2(h1, W2, asd2)
    w2e, den2 = _edge_w(asad2_l[:, 0:16], asad2_l[:, 16:32], srcw, dstw)
    num2 = _msg2(*[xw2_ch[c] for c in range(2)], srcw, dstw, w2e)
    h2 = _epi2(num2, den2, m2, b2.reshape(1, -1), g2.reshape(1, -1),
               be2.reshape(1, -1))
    return h2[:N_NODES]


# speedup vs baseline: 6.7498x; 1.8194x over previous
import functools

import jax
import jax.numpy as jnp
from jax import lax
from jax.experimental import pallas as pl
from jax.experimental.pallas import tpu as pltpu
from jax.experimental.pallas import tpu_sc as plsc

N_NODES = 10000
N_EDGES = 160000
D = 256
HID = 256
HEADS = 8

NC = 2
NS = 16
NW = NC * NS
L = 16

NPAD = 10240
EPAD = 172032
KB = 64
GRP = 3
NB_W = EPAD // NW // KB
ROWS_W = NPAD // NW
STRIPE = NPAD // NS

_MESH = plsc.VectorSubcoreMesh(core_axis_name="c", subcore_axis_name="s")
_SC_PARAMS = pltpu.CompilerParams(use_tc_tiling_on_sc=False,
                                  needs_layout_passes=False)


def _gather_body(table_hbm, idx_hbm, out_hbm, idx_v, rows_v, sem):
    cid = lax.axis_index("c")
    sid = lax.axis_index("s")
    wid = sid * NC + cid
    pltpu.sync_copy(idx_hbm.at[wid], idx_v)
    descs = []
    for j in range(ROWS_W // KB):
        descs.append(
            pltpu.async_copy(table_hbm.at[idx_v.at[j]],
                             rows_v.at[pl.ds(j * KB, KB)], sem))
    for d in descs:
        d.wait()
    pltpu.sync_copy(rows_v, out_hbm.at[pl.ds(wid * ROWS_W, ROWS_W)])


_gather_rows = pl.kernel(
    _gather_body,
    out_type=jax.ShapeDtypeStruct((NPAD, D), jnp.float32),
    mesh=_MESH,
    compiler_params=_SC_PARAMS,
    scratch_types=[
        pltpu.VMEM((ROWS_W // KB, KB), jnp.int32),
        pltpu.VMEM((ROWS_W, D), jnp.float32),
        pltpu.SemaphoreType.DMA,
    ],
)


def _edge_w_body(ta_hbm, tb_hbm, srcw_hbm, dstw_hbm, w_hbm, den_hbm,
                 idx_s, idx_d, buf_a, buf_b, buf_w, den_sh,
                 sem_g, sem_h, sem_s, sem_o):
    cid = lax.axis_index("c")
    sid = lax.axis_index("s")
    wid = sid * NC + cid
    pltpu.sync_copy(srcw_hbm.at[wid], idx_s)
    pltpu.sync_copy(dstw_hbm.at[wid], idx_d)

    def zr(r, _):
        buf_w[0, r, :] = jnp.zeros((L,), jnp.float32)
        return 0
    lax.fori_loop(0, KB, zr, 0)
    for t in range(STRIPE // KB):
        pltpu.sync_copy(buf_w.at[0], den_sh.at[pl.ds(sid * STRIPE + t * KB, KB)])
    plsc.subcore_barrier()

    def jb(g, _):
        gds = []
        for b in range(GRP):
            j = g * GRP + b
            gds.append((
                pltpu.async_copy(ta_hbm.at[idx_s.at[j]], buf_a.at[b], sem_g),
                pltpu.async_copy(tb_hbm.at[idx_d.at[j]], buf_b.at[b], sem_h)))
        sds = []
        for b in range(GRP):
            j = g * GRP + b
            gds[b][0].wait()
            gds[b][1].wait()

            def ek(k, _):
                e = buf_a[b, k, :] + buf_b[b, k, :]
                e = jnp.where(e >= 0.0, e, 0.2 * e)
                buf_w[b, k, :] = jnp.exp(e)
                return 0
            lax.fori_loop(0, KB, ek, 0)
            sds.append((
                pltpu.async_copy(buf_w.at[b], den_sh.at[idx_d.at[j]], sem_s,
                                 add=True),
                pltpu.async_copy(buf_w.at[b], w_hbm.at[wid].at[j], sem_o)))
        for b in range(GRP):
            sds[b][0].wait()
            sds[b][1].wait()
        return 0
    lax.fori_loop(0, NB_W // GRP, jb, 0)
    plsc.subcore_barrier()
    pltpu.sync_copy(den_sh.at[pl.ds(sid * STRIPE, STRIPE)],
                    den_hbm.at[cid].at[pl.ds(sid * STRIPE, STRIPE)])


_edge_w = pl.kernel(
    _edge_w_body,
    out_type=(
        jax.ShapeDtypeStruct((NW, NB_W, KB, L), jnp.float32),
        jax.ShapeDtypeStruct((NC, NPAD, L), jnp.float32),
    ),
    mesh=_MESH,
    compiler_params=_SC_PARAMS,
    scratch_types=[
        pltpu.VMEM((NB_W, KB), jnp.int32),
        pltpu.VMEM((NB_W, KB), jnp.int32),
        pltpu.VMEM((GRP, KB, L), jnp.float32),
        pltpu.VMEM((GRP, KB, L), jnp.float32),
        pltpu.VMEM((GRP, KB, L), jnp.float32),
        pltpu.VMEM_SHARED((NPAD, L), jnp.float32),
        pltpu.SemaphoreType.DMA,
        pltpu.SemaphoreType.DMA,
        pltpu.SemaphoreType.DMA,
        pltpu.SemaphoreType.DMA,
    ],
)


def _make_msg(nch, head_lanes):
    dtot = nch * 128

    def body(*refs):
        tabs = refs[:nch]
        srcw_hbm, dstw_hbm, w_hbm, out_hbm = refs[nch:nch + 4]
        idx_s, idx_d, rbuf, wbuf, zbuf, acc_sh, sem_g, sem_h, sem_s = \
            refs[nch + 4:]
        cid = lax.axis_index("c")
        sid = lax.axis_index("s")
        wid = sid * NC + cid
        pltpu.sync_copy(srcw_hbm.at[wid], idx_s)
        pltpu.sync_copy(dstw_hbm.at[wid], idx_d)

        def zr(r, _):
            for t in range(8):
                zbuf[r, pl.ds(t * L, L)] = jnp.zeros((L,), jnp.float32)
            return 0
        lax.fori_loop(0, KB, zr, 0)

        for ci in range(nch):
            hl = head_lanes[ci]
            for t in range(STRIPE // KB):
                pltpu.sync_copy(zbuf, acc_sh.at[pl.ds(sid * STRIPE + t * KB, KB)])
            plsc.subcore_barrier()

            def jb(g, _):
                gds = []
                for b in range(GRP):
                    j = g * GRP + b
                    gds.append((
                        pltpu.async_copy(tabs[ci].at[idx_s.at[j]],
                                         rbuf.at[b], sem_g),
                        pltpu.async_copy(w_hbm.at[wid].at[j],
                                         wbuf.at[b], sem_h)))
                sds = []
                for b in range(GRP):
                    j = g * GRP + b
                    gds[b][0].wait()
                    gds[b][1].wait()

                    def kb_(k, _):
                        wv = plsc.load_gather(
                            wbuf.at[b],
                            [jnp.full((L,), k, jnp.int32),
                             jnp.full((L,), hl, jnp.int32)])
                        for t in range(8):
                            rbuf[b, k, pl.ds(t * L, L)] = (
                                rbuf[b, k, pl.ds(t * L, L)] * wv)
                        return 0
                    lax.fori_loop(0, KB, kb_, 0)
                    sds.append(pltpu.async_copy(
                        rbuf.at[b], acc_sh.at[idx_d.at[j]], sem_s, add=True))
                for b in range(GRP):
                    sds[b].wait()
                return 0
            lax.fori_loop(0, NB_W // GRP, jb, 0)
            plsc.subcore_barrier()
            pltpu.sync_copy(
                acc_sh.at[pl.ds(sid * STRIPE, STRIPE)],
                out_hbm.at[cid].at[pl.ds(sid * STRIPE, STRIPE),
                                   pl.ds(ci * 128, 128)])
            plsc.subcore_barrier()

    return pl.kernel(
        body,
        out_type=jax.ShapeDtypeStruct((NC, NPAD, dtot), jnp.float32),
        mesh=_MESH,
        compiler_params=_SC_PARAMS,
        scratch_types=[
            pltpu.VMEM((NB_W, KB), jnp.int32),
            pltpu.VMEM((NB_W, KB), jnp.int32),
            pltpu.VMEM((GRP, KB, 128), jnp.float32),
            pltpu.VMEM((GRP, KB, L), jnp.float32),
            pltpu.VMEM((KB, 128), jnp.float32),
            pltpu.VMEM_SHARED((NPAD, 128), jnp.float32),
            pltpu.SemaphoreType.DMA,
            pltpu.SemaphoreType.DMA,
            pltpu.SemaphoreType.DMA,
        ],
    )


_msg16 = _make_msg(16, [c // 2 for c in range(16)])
_msg2 = _make_msg(2, [0, 0])


def _mm1_body(x_ref, w_ref, asd_ref, xw_ref, la_ref):
    c = pl.program_id(1)
    acc = jnp.dot(x_ref[...], w_ref[...], preferred_element_type=jnp.float32)
    xw_ref[...] = acc.reshape(1, 256, 128)
    la = jnp.dot(acc, asd_ref[...], preferred_element_type=jnp.float32)

    @pl.when(c == 0)
    def _():
        la_ref[...] = la

    @pl.when(c != 0)
    def _():
        la_ref[...] = la_ref[...] + la


def _mm1(x, w1, asd1):
    return pl.pallas_call(
        _mm1_body,
        grid=(NPAD // 256, 16),
        in_specs=[
            pl.BlockSpec((256, D), lambda i, c: (i, 0)),
            pl.BlockSpec((D, 128), lambda i, c: (0, c)),
            pl.BlockSpec((128, 128), lambda i, c: (c, 0)),
        ],
        out_specs=[
            pl.BlockSpec((1, 256, 128), lambda i, c: (c, i, 0)),
            pl.BlockSpec((256, 128), lambda i, c: (i, 0)),
        ],
        out_shape=[
            jax.ShapeDtypeStruct((16, NPAD, 128), jnp.float32),
            jax.ShapeDtypeStruct((NPAD, 128), jnp.float32),
        ],
    )(x, w1, asd1)


def _mm2_body(h_ref, w_ref, asd_ref, xw_ref, la_ref):
    c = pl.program_id(1)
    acc = jnp.dot(h_ref[...], w_ref[...], preferred_element_type=jnp.float32)
    xw_ref[...] = acc.reshape(1, 256, 128)
    la = jnp.dot(acc, asd_ref[...], preferred_element_type=jnp.float32)

    @pl.when(c == 0)
    def _():
        la_ref[...] = la

    @pl.when(c != 0)
    def _():
        la_ref[...] = la_ref[...] + la


def _mm2(h, w2, asd2):
    return pl.pallas_call(
        _mm2_body,
        grid=(NPAD // 256, 2),
        in_specs=[
            pl.BlockSpec((256, HEADS * HID), lambda i, c: (i, 0)),
            pl.BlockSpec((HEADS * HID, 128), lambda i, c: (0, c)),
            pl.BlockSpec((128, 128), lambda i, c: (c, 0)),
        ],
        out_specs=[
            pl.BlockSpec((1, 256, 128), lambda i, c: (c, i, 0)),
            pl.BlockSpec((256, 128), lambda i, c: (i, 0)),
        ],
        out_shape=[
            jax.ShapeDtypeStruct((2, NPAD, 128), jnp.float32),
            jax.ShapeDtypeStruct((NPAD, 128), jnp.float32),
        ],
    )(h, w2, asd2)


def _epi_body(num_ref, den_ref, m_ref, b_ref, g_ref, be_ref, out_ref):
    den = den_ref[0] + den_ref[1]
    dexp = jnp.dot(den, m_ref[...], preferred_element_type=jnp.float32)
    num = num_ref[0] + num_ref[1]
    h = num / (dexp + 1e-16) + b_ref[...]
    mu = jnp.mean(h, axis=-1, keepdims=True)
    var = jnp.mean((h - mu) ** 2, axis=-1, keepdims=True)
    hn = (h - mu) / jnp.sqrt(var + 1e-5) * g_ref[...] + be_ref[...]
    out_ref[...] = jnp.where(hn > 0.0, hn, jnp.exp(hn) - 1.0)


def _make_epi(dt):
    def run(num, den, m, b, g, be):
        return pl.pallas_call(
            _epi_body,
            grid=(NPAD // 256,),
            in_specs=[
                pl.BlockSpec((2, 256, dt), lambda i: (0, i, 0)),
                pl.BlockSpec((2, 256, L), lambda i: (0, i, 0)),
                pl.BlockSpec((L, dt), lambda i: (0, 0)),
                pl.BlockSpec((1, dt), lambda i: (0, 0)),
                pl.BlockSpec((1, dt), lambda i: (0, 0)),
                pl.BlockSpec((1, dt), lambda i: (0, 0)),
            ],
            out_specs=pl.BlockSpec((256, dt), lambda i: (i, 0)),
            out_shape=jax.ShapeDtypeStruct((NPAD, dt), jnp.float32),
        )(num, den, m, b, g, be)
    return run


_epi1 = _make_epi(HEADS * HID)
_epi2 = _make_epi(D)


def kernel(entity_ids, edge_index, entity_table,
           W1, a_src1, a_dst1, b1, g1, be1,
           W2, a_src2, a_dst2, b2, g2, be2):
    f32 = jnp.float32
    ids = entity_ids.astype(jnp.int32)
    pad_ids = jnp.concatenate(
        [ids, jnp.zeros((NPAD - N_NODES,), jnp.int32)]).reshape(
            NW, ROWS_W // KB, KB)
    loop = jnp.arange(N_NODES, dtype=jnp.int32)
    padc = jnp.full((EPAD - N_EDGES - N_NODES,), NPAD - 1, jnp.int32)
    src = jnp.concatenate([edge_index[0].astype(jnp.int32), loop, padc])
    dst = jnp.concatenate([edge_index[1].astype(jnp.int32), loop, padc])
    srcw = src.reshape(NW, NB_W, KB)
    dstw = dst.reshape(NW, NB_W, KB)

    eye8 = jnp.eye(HEADS, dtype=f32)
    as_mat1 = (eye8[:, None, :] * a_src1[:, :, None]).reshape(HEADS * HID, HEADS)
    ad_mat1 = (eye8[:, None, :] * a_dst1[:, :, None]).reshape(HEADS * HID, HEADS)
    z8 = jnp.zeros((HEADS * HID, 8), f32)
    asd1 = jnp.concatenate(
        [as_mat1, z8, ad_mat1, jnp.zeros((HEADS * HID, 128 - 24), f32)], axis=1)
    asd2 = jnp.concatenate(
        [a_src2.T, jnp.zeros((D, 15), f32),
         a_dst2.T, jnp.zeros((D, 128 - 17), f32)], axis=1)
    m1 = jnp.concatenate(
        [jnp.kron(eye8, jnp.ones((1, HID), f32)),
         jnp.zeros((8, HEADS * HID), f32)], axis=0)
    m2 = jnp.concatenate([jnp.ones((1, D), f32), jnp.zeros((15, D), f32)], axis=0)

    x = _gather_rows(entity_table, pad_ids)
    xw_ch, asad1 = _mm1(x, W1, asd1)
    w1e, den1 = _edge_w(asad1[:, 0:16], asad1[:, 16:32], srcw, dstw)
    num1 = _msg16(*[xw_ch[c] for c in range(16)], srcw, dstw, w1e)
    h1 = _epi1(num1, den1, m1, b1.reshape(1, -1), g1.reshape(1, -1),
               be1.reshape(1, -1))
    xw2_ch, asad2_l = _mm2(h1, W2, asd2)
    w2e, den2 = _edge_w(asad2_l[:, 0:16], asad2_l[:, 16:32], srcw, dstw)
    num2 = _msg2(*[xw2_ch[c] for c in range(2)], srcw, dstw, w2e)
    h2 = _epi2(num2, den2, m2, b2.reshape(1, -1), g2.reshape(1, -1),
               be2.reshape(1, -1))
    return h2[:N_NODES]


# speedup vs baseline: 6.9887x; 1.0354x over previous
import functools

import jax
import jax.numpy as jnp
from jax import lax
from jax.experimental import pallas as pl
from jax.experimental.pallas import tpu as pltpu
from jax.experimental.pallas import tpu_sc as plsc

N_NODES = 10000
N_EDGES = 160000
D = 256
HID = 256
HEADS = 8

NC = 2
NS = 16
NW = NC * NS
L = 16

NPAD = 10240
EPAD = 172032
KB = 64
GRP = 3
NB_W = EPAD // NW // KB
ROWS_W = NPAD // NW
STRIPE = NPAD // NS

_MESH = plsc.VectorSubcoreMesh(core_axis_name="c", subcore_axis_name="s")
_SC_PARAMS = pltpu.CompilerParams(use_tc_tiling_on_sc=False,
                                  needs_layout_passes=False)


def _gather_body(table_hbm, idx_hbm, out_hbm, idx_v, rows_v, sem):
    cid = lax.axis_index("c")
    sid = lax.axis_index("s")
    wid = sid * NC + cid
    pltpu.sync_copy(idx_hbm.at[wid], idx_v)
    descs = []
    for j in range(ROWS_W // KB):
        descs.append(
            pltpu.async_copy(table_hbm.at[idx_v.at[j]],
                             rows_v.at[pl.ds(j * KB, KB)], sem))
    for d in descs:
        d.wait()
    pltpu.sync_copy(rows_v, out_hbm.at[pl.ds(wid * ROWS_W, ROWS_W)])


_gather_rows = pl.kernel(
    _gather_body,
    out_type=jax.ShapeDtypeStruct((NPAD, D), jnp.float32),
    mesh=_MESH,
    compiler_params=_SC_PARAMS,
    scratch_types=[
        pltpu.VMEM((ROWS_W // KB, KB), jnp.int32),
        pltpu.VMEM((ROWS_W, D), jnp.float32),
        pltpu.SemaphoreType.DMA,
    ],
)


def _edge_w_body(ta_hbm, tb_hbm, srcw_hbm, dstw_hbm, w_hbm, den_hbm,
                 idx_s, idx_d, buf_a, buf_b, buf_w, den_sh,
                 sem_g, sem_h, sem_s, sem_o):
    cid = lax.axis_index("c")
    sid = lax.axis_index("s")
    wid = sid * NC + cid
    pltpu.sync_copy(srcw_hbm.at[wid], idx_s)
    pltpu.sync_copy(dstw_hbm.at[wid], idx_d)

    def zr(r, _):
        buf_w[0, r, :] = jnp.zeros((L,), jnp.float32)
        return 0
    lax.fori_loop(0, KB, zr, 0)
    for t in range(STRIPE // KB):
        pltpu.sync_copy(buf_w.at[0], den_sh.at[pl.ds(sid * STRIPE + t * KB, KB)])
    plsc.subcore_barrier()

    def jb(g, _):
        gds = []
        for b in range(GRP):
            j = g * GRP + b
            gds.append((
                pltpu.async_copy(ta_hbm.at[idx_s.at[j]], buf_a.at[b], sem_g),
                pltpu.async_copy(tb_hbm.at[idx_d.at[j]], buf_b.at[b], sem_h)))
        sds = []
        for b in range(GRP):
            j = g * GRP + b
            gds[b][0].wait()
            gds[b][1].wait()

            def ek(k, _):
                e = buf_a[b, k, :] + buf_b[b, k, :]
                e = jnp.where(e >= 0.0, e, 0.2 * e)
                buf_w[b, k, :] = jnp.exp(e)
                return 0
            lax.fori_loop(0, KB, ek, 0)
            sds.append((
                pltpu.async_copy(buf_w.at[b], den_sh.at[idx_d.at[j]], sem_s,
                                 add=True),
                pltpu.async_copy(buf_w.at[b], w_hbm.at[wid].at[j], sem_o)))
        for b in range(GRP):
            sds[b][0].wait()
            sds[b][1].wait()
        return 0
    lax.fori_loop(0, NB_W // GRP, jb, 0)
    plsc.subcore_barrier()
    pltpu.sync_copy(den_sh.at[pl.ds(sid * STRIPE, STRIPE)],
                    den_hbm.at[cid].at[pl.ds(sid * STRIPE, STRIPE)])


_edge_w = pl.kernel(
    _edge_w_body,
    out_type=(
        jax.ShapeDtypeStruct((NW, NB_W, KB, L), jnp.float32),
        jax.ShapeDtypeStruct((NC, NPAD, L), jnp.float32),
    ),
    mesh=_MESH,
    compiler_params=_SC_PARAMS,
    scratch_types=[
        pltpu.VMEM((NB_W, KB), jnp.int32),
        pltpu.VMEM((NB_W, KB), jnp.int32),
        pltpu.VMEM((GRP, KB, L), jnp.float32),
        pltpu.VMEM((GRP, KB, L), jnp.float32),
        pltpu.VMEM((GRP, KB, L), jnp.float32),
        pltpu.VMEM_SHARED((NPAD, L), jnp.float32),
        pltpu.SemaphoreType.DMA,
        pltpu.SemaphoreType.DMA,
        pltpu.SemaphoreType.DMA,
        pltpu.SemaphoreType.DMA,
    ],
)


def _make_msg(nch, per_head):

    def body(xw_st, srcw_hbm, dstw_hbm, w_hbm, out_hbm,
             idx_s, idx_d, rbuf, wbuf, zbuf, acc_sh, sem_g, sem_h, sem_s):
        cid = lax.axis_index("c")
        sid = lax.axis_index("s")
        wid = sid * NC + cid
        pltpu.sync_copy(srcw_hbm.at[wid], idx_s)
        pltpu.sync_copy(dstw_hbm.at[wid], idx_d)

        def zr(r, _):
            for t in range(8):
                zbuf[r, pl.ds(t * L, L)] = jnp.zeros((L,), jnp.float32)
            return 0
        lax.fori_loop(0, KB, zr, 0)

        def chunk_body(ci, _):
            if per_head:
                hlv = jnp.full((L,), ci // 2, jnp.int32)
            else:
                hlv = jnp.zeros((L,), jnp.int32)
            for t in range(STRIPE // KB):
                pltpu.sync_copy(zbuf, acc_sh.at[pl.ds(sid * STRIPE + t * KB, KB)])
            plsc.subcore_barrier()

            def jb(g, _):
                gds = []
                for b in range(GRP):
                    j = g * GRP + b
                    gds.append((
                        pltpu.async_copy(xw_st.at[ci].at[idx_s.at[j]],
                                         rbuf.at[b], sem_g),
                        pltpu.async_copy(w_hbm.at[wid].at[j],
                                         wbuf.at[b], sem_h)))
                sds = []
                for b in range(GRP):
                    j = g * GRP + b
                    gds[b][0].wait()
                    gds[b][1].wait()

                    def kb8(kk, _):
                        for e in range(8):
                            k = kk * 8 + e
                            wv = plsc.load_gather(
                                wbuf.at[b],
                                [jnp.full((L,), k, jnp.int32), hlv])
                            for t in range(8):
                                rbuf[b, k, pl.ds(t * L, L)] = (
                                    rbuf[b, k, pl.ds(t * L, L)] * wv)
                        return 0
                    lax.fori_loop(0, KB // 8, kb8, 0)
                    sds.append(pltpu.async_copy(
                        rbuf.at[b], acc_sh.at[idx_d.at[j]], sem_s, add=True))
                for b in range(GRP):
                    sds[b].wait()
                return 0
            lax.fori_loop(0, NB_W // GRP, jb, 0)
            plsc.subcore_barrier()
            pltpu.sync_copy(
                acc_sh.at[pl.ds(sid * STRIPE, STRIPE)],
                out_hbm.at[cid].at[pl.ds(sid * STRIPE, STRIPE),
                                   pl.ds(ci * 128, 128)])
            plsc.subcore_barrier()
            return 0
        lax.fori_loop(0, nch, chunk_body, 0)

    return pl.kernel(
        body,
        out_type=jax.ShapeDtypeStruct((NC, NPAD, nch * 128), jnp.float32),
        mesh=_MESH,
        compiler_params=_SC_PARAMS,
        scratch_types=[
            pltpu.VMEM((NB_W, KB), jnp.int32),
            pltpu.VMEM((NB_W, KB), jnp.int32),
            pltpu.VMEM((GRP, KB, 128), jnp.float32),
            pltpu.VMEM((GRP, KB, L), jnp.float32),
            pltpu.VMEM((KB, 128), jnp.float32),
            pltpu.VMEM_SHARED((NPAD, 128), jnp.float32),
            pltpu.SemaphoreType.DMA,
            pltpu.SemaphoreType.DMA,
            pltpu.SemaphoreType.DMA,
        ],
    )


_msg16 = _make_msg(16, True)
_msg2 = _make_msg(2, False)


def _mm1_body(x_ref, w_ref, asd_ref, xw_ref, la_ref):
    c = pl.program_id(1)
    acc = jnp.dot(x_ref[...], w_ref[...], preferred_element_type=jnp.float32)
    xw_ref[...] = acc.reshape(1, 256, 128)
    la = jnp.dot(acc, asd_ref[...], preferred_element_type=jnp.float32)

    @pl.when(c == 0)
    def _():
        la_ref[...] = la

    @pl.when(c != 0)
    def _():
        la_ref[...] = la_ref[...] + la


def _mm1(x, w1, asd1):
    return pl.pallas_call(
        _mm1_body,
        grid=(NPAD // 256, 16),
        in_specs=[
            pl.BlockSpec((256, D), lambda i, c: (i, 0)),
            pl.BlockSpec((D, 128), lambda i, c: (0, c)),
            pl.BlockSpec((128, 128), lambda i, c: (c, 0)),
        ],
        out_specs=[
            pl.BlockSpec((1, 256, 128), lambda i, c: (c, i, 0)),
            pl.BlockSpec((256, 128), lambda i, c: (i, 0)),
        ],
        out_shape=[
            jax.ShapeDtypeStruct((16, NPAD, 128), jnp.float32),
            jax.ShapeDtypeStruct((NPAD, 128), jnp.float32),
        ],
    )(x, w1, asd1)


def _mm2_body(h_ref, w_ref, asd_ref, xw_ref, la_ref):
    c = pl.program_id(1)
    acc = jnp.dot(h_ref[...], w_ref[...], preferred_element_type=jnp.float32)
    xw_ref[...] = acc.reshape(1, 256, 128)
    la = jnp.dot(acc, asd_ref[...], preferred_element_type=jnp.float32)

    @pl.when(c == 0)
    def _():
        la_ref[...] = la

    @pl.when(c != 0)
    def _():
        la_ref[...] = la_ref[...] + la


def _mm2(h, w2, asd2):
    return pl.pallas_call(
        _mm2_body,
        grid=(NPAD // 256, 2),
        in_specs=[
            pl.BlockSpec((256, HEADS * HID), lambda i, c: (i, 0)),
            pl.BlockSpec((HEADS * HID, 128), lambda i, c: (0, c)),
            pl.BlockSpec((128, 128), lambda i, c: (c, 0)),
        ],
        out_specs=[
            pl.BlockSpec((1, 256, 128), lambda i, c: (c, i, 0)),
            pl.BlockSpec((256, 128), lambda i, c: (i, 0)),
        ],
        out_shape=[
            jax.ShapeDtypeStruct((2, NPAD, 128), jnp.float32),
            jax.ShapeDtypeStruct((NPAD, 128), jnp.float32),
        ],
    )(h, w2, asd2)


def _epi_body(num_ref, den_ref, m_ref, b_ref, g_ref, be_ref, out_ref):
    den = den_ref[0] + den_ref[1]
    dexp = jnp.dot(den, m_ref[...], preferred_element_type=jnp.float32)
    num = num_ref[0] + num_ref[1]
    h = num / (dexp + 1e-16) + b_ref[...]
    mu = jnp.mean(h, axis=-1, keepdims=True)
    var = jnp.mean((h - mu) ** 2, axis=-1, keepdims=True)
    hn = (h - mu) / jnp.sqrt(var + 1e-5) * g_ref[...] + be_ref[...]
    out_ref[...] = jnp.where(hn > 0.0, hn, jnp.exp(hn) - 1.0)


def _make_epi(dt):
    def run(num, den, m, b, g, be):
        return pl.pallas_call(
            _epi_body,
            grid=(NPAD // 256,),
            in_specs=[
                pl.BlockSpec((2, 256, dt), lambda i: (0, i, 0)),
                pl.BlockSpec((2, 256, L), lambda i: (0, i, 0)),
                pl.BlockSpec((L, dt), lambda i: (0, 0)),
                pl.BlockSpec((1, dt), lambda i: (0, 0)),
                pl.BlockSpec((1, dt), lambda i: (0, 0)),
                pl.BlockSpec((1, dt), lambda i: (0, 0)),
            ],
            out_specs=pl.BlockSpec((256, dt), lambda i: (i, 0)),
            out_shape=jax.ShapeDtypeStruct((NPAD, dt), jnp.float32),
        )(num, den, m, b, g, be)
    return run


_epi1 = _make_epi(HEADS * HID)
_epi2 = _make_epi(D)


def kernel(entity_ids, edge_index, entity_table,
           W1, a_src1, a_dst1, b1, g1, be1,
           W2, a_src2, a_dst2, b2, g2, be2):
    f32 = jnp.float32
    ids = entity_ids.astype(jnp.int32)
    pad_ids = jnp.concatenate(
        [ids, jnp.zeros((NPAD - N_NODES,), jnp.int32)]).reshape(
            NW, ROWS_W // KB, KB)
    loop = jnp.arange(N_NODES, dtype=jnp.int32)
    padc = jnp.full((EPAD - N_EDGES - N_NODES,), NPAD - 1, jnp.int32)
    src = jnp.concatenate([edge_index[0].astype(jnp.int32), loop, padc])
    dst = jnp.concatenate([edge_index[1].astype(jnp.int32), loop, padc])
    srcw = src.reshape(NW, NB_W, KB)
    dstw = dst.reshape(NW, NB_W, KB)

    eye8 = jnp.eye(HEADS, dtype=f32)
    as_mat1 = (eye8[:, None, :] * a_src1[:, :, None]).reshape(HEADS * HID, HEADS)
    ad_mat1 = (eye8[:, None, :] * a_dst1[:, :, None]).reshape(HEADS * HID, HEADS)
    z8 = jnp.zeros((HEADS * HID, 8), f32)
    asd1 = jnp.concatenate(
        [as_mat1, z8, ad_mat1, jnp.zeros((HEADS * HID, 128 - 24), f32)], axis=1)
    asd2 = jnp.concatenate(
        [a_src2.T, jnp.zeros((D, 15), f32),
         a_dst2.T, jnp.zeros((D, 128 - 17), f32)], axis=1)
    m1 = jnp.concatenate(
        [jnp.kron(eye8, jnp.ones((1, HID), f32)),
         jnp.zeros((8, HEADS * HID), f32)], axis=0)
    m2 = jnp.concatenate([jnp.ones((1, D), f32), jnp.zeros((15, D), f32)], axis=0)

    x = _gather_rows(entity_table, pad_ids)
    xw_ch, asad1 = _mm1(x, W1, asd1)
    w1e, den1 = _edge_w(asad1[:, 0:16], asad1[:, 16:32], srcw, dstw)
    num1 = _msg16(xw_ch, srcw, dstw, w1e)
    h1 = _epi1(num1, den1, m1, b1.reshape(1, -1), g1.reshape(1, -1),
               be1.reshape(1, -1))
    xw2_ch, asad2_l = _mm2(h1, W2, asd2)
    w2e, den2 = _edge_w(asad2_l[:, 0:16], asad2_l[:, 16:32], srcw, dstw)
    num2 = _msg2(xw2_ch, srcw, dstw, w2e)
    h2 = _epi2(num2, den2, m2, b2.reshape(1, -1), g2.reshape(1, -1),
               be2.reshape(1, -1))
    return h2[:N_NODES]


# speedup vs baseline: 7.6128x; 1.0893x over previous
import functools

import jax
import jax.numpy as jnp
from jax import lax
from jax.experimental import pallas as pl
from jax.experimental.pallas import tpu as pltpu
from jax.experimental.pallas import tpu_sc as plsc

N_NODES = 10000
N_EDGES = 160000
D = 256
HID = 256
HEADS = 8

NC = 2
NS = 16
NW = NC * NS
L = 16

NPAD = 10240
EPAD = 172032
KB = 64
GRP = 3
NB_W = EPAD // NW // KB
ROWS_W = NPAD // NW
STRIPE = NPAD // NS

_MESH = plsc.VectorSubcoreMesh(core_axis_name="c", subcore_axis_name="s")
_SC_PARAMS = pltpu.CompilerParams(use_tc_tiling_on_sc=False,
                                  needs_layout_passes=False)


def _gather_body(table_hbm, idx_hbm, out_hbm, idx_v, rows_v, sem):
    cid = lax.axis_index("c")
    sid = lax.axis_index("s")
    wid = sid * NC + cid
    pltpu.sync_copy(idx_hbm.at[wid], idx_v)
    descs = []
    for j in range(ROWS_W // KB):
        descs.append(
            pltpu.async_copy(table_hbm.at[idx_v.at[j]],
                             rows_v.at[pl.ds(j * KB, KB)], sem))
    for d in descs:
        d.wait()
    pltpu.sync_copy(rows_v, out_hbm.at[pl.ds(wid * ROWS_W, ROWS_W)])


_gather_rows = pl.kernel(
    _gather_body,
    out_type=jax.ShapeDtypeStruct((NPAD, D), jnp.float32),
    mesh=_MESH,
    compiler_params=_SC_PARAMS,
    scratch_types=[
        pltpu.VMEM((ROWS_W // KB, KB), jnp.int32),
        pltpu.VMEM((ROWS_W, D), jnp.float32),
        pltpu.SemaphoreType.DMA,
    ],
)


def _edge_w_body(ta_hbm, tb_hbm, srcw_hbm, dstw_hbm, w_hbm, den_hbm,
                 idx_s, idx_d, buf_a, buf_b, buf_w, den_sh,
                 sem_g, sem_h, sem_s, sem_o):
    cid = lax.axis_index("c")
    sid = lax.axis_index("s")
    wid = sid * NC + cid
    pltpu.sync_copy(srcw_hbm.at[wid], idx_s)
    pltpu.sync_copy(dstw_hbm.at[wid], idx_d)

    def zr(r, _):
        buf_w[0, r, :] = jnp.zeros((L,), jnp.float32)
        return 0
    lax.fori_loop(0, KB, zr, 0)
    for t in range(STRIPE // KB):
        pltpu.sync_copy(buf_w.at[0], den_sh.at[pl.ds(sid * STRIPE + t * KB, KB)])
    plsc.subcore_barrier()

    def jb(g, _):
        gds = []
        for b in range(GRP):
            j = g * GRP + b
            gds.append((
                pltpu.async_copy(ta_hbm.at[idx_s.at[j]], buf_a.at[b], sem_g),
                pltpu.async_copy(tb_hbm.at[idx_d.at[j]], buf_b.at[b], sem_h)))
        sds = []
        for b in range(GRP):
            j = g * GRP + b
            gds[b][0].wait()
            gds[b][1].wait()

            def ek(k, _):
                e = buf_a[b, k, :] + buf_b[b, k, :]
                e = jnp.where(e >= 0.0, e, 0.2 * e)
                buf_w[b, k, :] = jnp.exp(e)
                return 0
            lax.fori_loop(0, KB, ek, 0)
            sds.append((
                pltpu.async_copy(buf_w.at[b], den_sh.at[idx_d.at[j]], sem_s,
                                 add=True),
                pltpu.async_copy(buf_w.at[b], w_hbm.at[wid].at[j], sem_o)))
        for b in range(GRP):
            sds[b][0].wait()
            sds[b][1].wait()
        return 0
    lax.fori_loop(0, NB_W // GRP, jb, 0)
    plsc.subcore_barrier()
    pltpu.sync_copy(den_sh.at[pl.ds(sid * STRIPE, STRIPE)],
                    den_hbm.at[cid].at[pl.ds(sid * STRIPE, STRIPE)])


_edge_w = pl.kernel(
    _edge_w_body,
    out_type=(
        jax.ShapeDtypeStruct((NW, NB_W, KB, L), jnp.float32),
        jax.ShapeDtypeStruct((NC, NPAD, L), jnp.float32),
    ),
    mesh=_MESH,
    compiler_params=_SC_PARAMS,
    scratch_types=[
        pltpu.VMEM((NB_W, KB), jnp.int32),
        pltpu.VMEM((NB_W, KB), jnp.int32),
        pltpu.VMEM((GRP, KB, L), jnp.float32),
        pltpu.VMEM((GRP, KB, L), jnp.float32),
        pltpu.VMEM((GRP, KB, L), jnp.float32),
        pltpu.VMEM_SHARED((NPAD, L), jnp.float32),
        pltpu.SemaphoreType.DMA,
        pltpu.SemaphoreType.DMA,
        pltpu.SemaphoreType.DMA,
        pltpu.SemaphoreType.DMA,
    ],
)


NH = 2

def _make_msg(nch, per_head, s0):

    def body(xw_st, srcs_hbm, dsts_hbm, w_hbm, out_hbm,
             idx_s, idx_d, rbuf, wbuf, zbuf, acc_sh, sem_g, sem_h, sem_s):
        cid = lax.axis_index("c")
        sid = lax.axis_index("s")

        def zr(r, _):
            for t in range(8):
                zbuf[r, pl.ds(t * L, L)] = jnp.zeros((L,), jnp.float32)
            return 0
        lax.fori_loop(0, KB, zr, 0)

        clo = jnp.where(cid == 0, 0, s0)
        chi = jnp.where(cid == 0, s0, nch)

        def chunk_body(ci, _):
            if per_head:
                hlv = jnp.full((L,), ci // 2, jnp.int32)
            else:
                hlv = jnp.zeros((L,), jnp.int32)
            for t in range(STRIPE // KB):
                pltpu.sync_copy(zbuf, acc_sh.at[pl.ds(sid * STRIPE + t * KB, KB)])
            plsc.subcore_barrier()

            for half in range(NH):
                pltpu.sync_copy(srcs_hbm.at[sid].at[half], idx_s)
                pltpu.sync_copy(dsts_hbm.at[sid].at[half], idx_d)

                def jb(g, _):
                    gds = []
                    for b in range(GRP):
                        j = g * GRP + b
                        gds.append((
                            pltpu.async_copy(xw_st.at[ci].at[idx_s.at[j]],
                                             rbuf.at[b], sem_g),
                            pltpu.async_copy(w_hbm.at[sid].at[half].at[j],
                                             wbuf.at[b], sem_h)))
                    sds = []
                    for b in range(GRP):
                        j = g * GRP + b
                        gds[b][0].wait()
                        gds[b][1].wait()

                        def kb8(kk, _):
                            for e in range(8):
                                k = kk * 8 + e
                                wv = plsc.load_gather(
                                    wbuf.at[b],
                                    [jnp.full((L,), k, jnp.int32), hlv])
                                for t in range(8):
                                    rbuf[b, k, pl.ds(t * L, L)] = (
                                        rbuf[b, k, pl.ds(t * L, L)] * wv)
                            return 0
                        lax.fori_loop(0, KB // 8, kb8, 0)
                        sds.append(pltpu.async_copy(
                            rbuf.at[b], acc_sh.at[idx_d.at[j]], sem_s,
                            add=True))
                    for b in range(GRP):
                        sds[b].wait()
                    return 0
                lax.fori_loop(0, NB_W // GRP, jb, 0)
            plsc.subcore_barrier()
            pltpu.sync_copy(
                acc_sh.at[pl.ds(sid * STRIPE, STRIPE)],
                out_hbm.at[pl.ds(sid * STRIPE, STRIPE),
                           pl.ds(ci * 128, 128)])
            plsc.subcore_barrier()
            return 0
        lax.fori_loop(clo, chi, chunk_body, 0)

    return pl.kernel(
        body,
        out_type=jax.ShapeDtypeStruct((NPAD, nch * 128), jnp.float32),
        mesh=_MESH,
        compiler_params=_SC_PARAMS,
        scratch_types=[
            pltpu.VMEM((NB_W, KB), jnp.int32),
            pltpu.VMEM((NB_W, KB), jnp.int32),
            pltpu.VMEM((GRP, KB, 128), jnp.float32),
            pltpu.VMEM((GRP, KB, L), jnp.float32),
            pltpu.VMEM((KB, 128), jnp.float32),
            pltpu.VMEM_SHARED((NPAD, 128), jnp.float32),
            pltpu.SemaphoreType.DMA,
            pltpu.SemaphoreType.DMA,
            pltpu.SemaphoreType.DMA,
        ],
    )


_msg16 = _make_msg(16, True, 6)
_msg2 = _make_msg(2, False, 1)


def _mm1_body(x_ref, w_ref, asd_ref, xw_ref, la_ref):
    c = pl.program_id(1)
    acc = jnp.dot(x_ref[...], w_ref[...], preferred_element_type=jnp.float32)
    xw_ref[...] = acc.reshape(1, 256, 128)
    la = jnp.dot(acc, asd_ref[...], preferred_element_type=jnp.float32)

    @pl.when(c == 0)
    def _():
        la_ref[...] = la

    @pl.when(c != 0)
    def _():
        la_ref[...] = la_ref[...] + la


def _mm1(x, w1, asd1):
    return pl.pallas_call(
        _mm1_body,
        grid=(NPAD // 256, 16),
        in_specs=[
            pl.BlockSpec((256, D), lambda i, c: (i, 0)),
            pl.BlockSpec((D, 128), lambda i, c: (0, c)),
            pl.BlockSpec((128, 128), lambda i, c: (c, 0)),
        ],
        out_specs=[
            pl.BlockSpec((1, 256, 128), lambda i, c: (c, i, 0)),
            pl.BlockSpec((256, 128), lambda i, c: (i, 0)),
        ],
        out_shape=[
            jax.ShapeDtypeStruct((16, NPAD, 128), jnp.float32),
            jax.ShapeDtypeStruct((NPAD, 128), jnp.float32),
        ],
    )(x, w1, asd1)


def _mm2_body(h_ref, w_ref, asd_ref, xw_ref, la_ref):
    c = pl.program_id(1)
    acc = jnp.dot(h_ref[...], w_ref[...], preferred_element_type=jnp.float32)
    xw_ref[...] = acc.reshape(1, 256, 128)
    la = jnp.dot(acc, asd_ref[...], preferred_element_type=jnp.float32)

    @pl.when(c == 0)
    def _():
        la_ref[...] = la

    @pl.when(c != 0)
    def _():
        la_ref[...] = la_ref[...] + la


def _mm2(h, w2, asd2):
    return pl.pallas_call(
        _mm2_body,
        grid=(NPAD // 256, 2),
        in_specs=[
            pl.BlockSpec((256, HEADS * HID), lambda i, c: (i, 0)),
            pl.BlockSpec((HEADS * HID, 128), lambda i, c: (0, c)),
            pl.BlockSpec((128, 128), lambda i, c: (c, 0)),
        ],
        out_specs=[
            pl.BlockSpec((1, 256, 128), lambda i, c: (c, i, 0)),
            pl.BlockSpec((256, 128), lambda i, c: (i, 0)),
        ],
        out_shape=[
            jax.ShapeDtypeStruct((2, NPAD, 128), jnp.float32),
            jax.ShapeDtypeStruct((NPAD, 128), jnp.float32),
        ],
    )(h, w2, asd2)


def _epi_body(num_ref, den_ref, m_ref, b_ref, g_ref, be_ref, out_ref):
    den = den_ref[0] + den_ref[1]
    dexp = jnp.dot(den, m_ref[...], preferred_element_type=jnp.float32)
    h = num_ref[...] / (dexp + 1e-16) + b_ref[...]
    mu = jnp.mean(h, axis=-1, keepdims=True)
    var = jnp.mean((h - mu) ** 2, axis=-1, keepdims=True)
    hn = (h - mu) / jnp.sqrt(var + 1e-5) * g_ref[...] + be_ref[...]
    out_ref[...] = jnp.where(hn > 0.0, hn, jnp.exp(hn) - 1.0)


def _make_epi(dt):
    def run(num, den, m, b, g, be):
        return pl.pallas_call(
            _epi_body,
            grid=(NPAD // 256,),
            in_specs=[
                pl.BlockSpec((256, dt), lambda i: (i, 0)),
                pl.BlockSpec((2, 256, L), lambda i: (0, i, 0)),
                pl.BlockSpec((L, dt), lambda i: (0, 0)),
                pl.BlockSpec((1, dt), lambda i: (0, 0)),
                pl.BlockSpec((1, dt), lambda i: (0, 0)),
                pl.BlockSpec((1, dt), lambda i: (0, 0)),
            ],
            out_specs=pl.BlockSpec((256, dt), lambda i: (i, 0)),
            out_shape=jax.ShapeDtypeStruct((NPAD, dt), jnp.float32),
        )(num, den, m, b, g, be)
    return run


_epi1 = _make_epi(HEADS * HID)
_epi2 = _make_epi(D)


def kernel(entity_ids, edge_index, entity_table,
           W1, a_src1, a_dst1, b1, g1, be1,
           W2, a_src2, a_dst2, b2, g2, be2):
    f32 = jnp.float32
    ids = entity_ids.astype(jnp.int32)
    pad_ids = jnp.concatenate(
        [ids, jnp.zeros((NPAD - N_NODES,), jnp.int32)]).reshape(
            NW, ROWS_W // KB, KB)
    loop = jnp.arange(N_NODES, dtype=jnp.int32)
    padc = jnp.full((EPAD - N_EDGES - N_NODES,), NPAD - 1, jnp.int32)
    src = jnp.concatenate([edge_index[0].astype(jnp.int32), loop, padc])
    dst = jnp.concatenate([edge_index[1].astype(jnp.int32), loop, padc])
    srcw = src.reshape(NW, NB_W, KB)
    dstw = dst.reshape(NW, NB_W, KB)
    srcs = src.reshape(NS, NH, NB_W, KB)
    dsts = dst.reshape(NS, NH, NB_W, KB)

    eye8 = jnp.eye(HEADS, dtype=f32)
    as_mat1 = (eye8[:, None, :] * a_src1[:, :, None]).reshape(HEADS * HID, HEADS)
    ad_mat1 = (eye8[:, None, :] * a_dst1[:, :, None]).reshape(HEADS * HID, HEADS)
    z8 = jnp.zeros((HEADS * HID, 8), f32)
    asd1 = jnp.concatenate(
        [as_mat1, z8, ad_mat1, jnp.zeros((HEADS * HID, 128 - 24), f32)], axis=1)
    asd2 = jnp.concatenate(
        [a_src2.T, jnp.zeros((D, 15), f32),
         a_dst2.T, jnp.zeros((D, 128 - 17), f32)], axis=1)
    m1 = jnp.concatenate(
        [jnp.kron(eye8, jnp.ones((1, HID), f32)),
         jnp.zeros((8, HEADS * HID), f32)], axis=0)
    m2 = jnp.concatenate([jnp.ones((1, D), f32), jnp.zeros((15, D), f32)], axis=0)

    x = _gather_rows(entity_table, pad_ids)
    xw_ch, asad1 = _mm1(x, W1, asd1)
    w1e, den1 = _edge_w(asad1[:, 0:16], asad1[:, 16:32], srcw, dstw)
    num1 = _msg16(xw_ch, srcs, dsts, w1e.reshape(NS, NH, NB_W, KB, L))
    h1 = _epi1(num1, den1, m1, b1.reshape(1, -1), g1.reshape(1, -1),
               be1.reshape(1, -1))
    xw2_ch, asad2_l = _mm2(h1, W2, asd2)
    w2e, den2 = _edge_w(asad2_l[:, 0:16], asad2_l[:, 16:32], srcw, dstw)
    num2 = _msg2(xw2_ch, srcs, dsts, w2e.reshape(NS, NH, NB_W, KB, L))
    h2 = _epi2(num2, den2, m2, b2.reshape(1, -1), g2.reshape(1, -1),
               be2.reshape(1, -1))
    return h2[:N_NODES]


# speedup vs baseline: 7.6754x; 1.0082x over previous
import functools

import jax
import jax.numpy as jnp
from jax import lax
from jax.experimental import pallas as pl
from jax.experimental.pallas import tpu as pltpu
from jax.experimental.pallas import tpu_sc as plsc

N_NODES = 10000
N_EDGES = 160000
D = 256
HID = 256
HEADS = 8

NC = 2
NS = 16
NW = NC * NS
L = 16

NPAD = 10240
EPAD = 172032
KB = 64
GRP = 3
NB_W = EPAD // NW // KB
ROWS_W = NPAD // NW
STRIPE = NPAD // NS

_MESH = plsc.VectorSubcoreMesh(core_axis_name="c", subcore_axis_name="s")
_SC_PARAMS = pltpu.CompilerParams(use_tc_tiling_on_sc=False,
                                  needs_layout_passes=False)


def _gather_body(table_hbm, idx_hbm, out_hbm, idx_v, rows_v, sem):
    cid = lax.axis_index("c")
    sid = lax.axis_index("s")
    wid = sid * NC + cid
    pltpu.sync_copy(idx_hbm.at[wid], idx_v)
    descs = []
    for j in range(ROWS_W // KB):
        descs.append(
            pltpu.async_copy(table_hbm.at[idx_v.at[j]],
                             rows_v.at[pl.ds(j * KB, KB)], sem))
    for d in descs:
        d.wait()
    pltpu.sync_copy(rows_v, out_hbm.at[pl.ds(wid * ROWS_W, ROWS_W)])


_gather_rows = pl.kernel(
    _gather_body,
    out_type=jax.ShapeDtypeStruct((NPAD, D), jnp.float32),
    mesh=_MESH,
    compiler_params=_SC_PARAMS,
    scratch_types=[
        pltpu.VMEM((ROWS_W // KB, KB), jnp.int32),
        pltpu.VMEM((ROWS_W, D), jnp.float32),
        pltpu.SemaphoreType.DMA,
    ],
)


def _edge_w_body(ta_hbm, tb_hbm, srcw_hbm, dstw_hbm, w_hbm, den_hbm,
                 idx_s, idx_d, buf_a, buf_b, buf_w, den_sh,
                 sem_g, sem_h, sem_s, sem_o):
    cid = lax.axis_index("c")
    sid = lax.axis_index("s")
    wid = sid * NC + cid
    pltpu.sync_copy(srcw_hbm.at[wid], idx_s)
    pltpu.sync_copy(dstw_hbm.at[wid], idx_d)

    def zr(r, _):
        buf_w[0, r, :] = jnp.zeros((L,), jnp.float32)
        return 0
    lax.fori_loop(0, KB, zr, 0)
    for t in range(STRIPE // KB):
        pltpu.sync_copy(buf_w.at[0], den_sh.at[pl.ds(sid * STRIPE + t * KB, KB)])
    plsc.subcore_barrier()

    def jb(g, _):
        gds = []
        for b in range(GRP):
            j = g * GRP + b
            gds.append((
                pltpu.async_copy(ta_hbm.at[idx_s.at[j]], buf_a.at[b], sem_g),
                pltpu.async_copy(tb_hbm.at[idx_d.at[j]], buf_b.at[b], sem_h)))
        sds = []
        for b in range(GRP):
            j = g * GRP + b
            gds[b][0].wait()
            gds[b][1].wait()

            def ek(k, _):
                e = buf_a[b, k, :] + buf_b[b, k, :]
                e = jnp.where(e >= 0.0, e, 0.2 * e)
                buf_w[b, k, :] = jnp.exp(e)
                return 0
            lax.fori_loop(0, KB, ek, 0)
            sds.append((
                pltpu.async_copy(buf_w.at[b], den_sh.at[idx_d.at[j]], sem_s,
                                 add=True),
                pltpu.async_copy(buf_w.at[b], w_hbm.at[wid].at[j], sem_o)))
        for b in range(GRP):
            sds[b][0].wait()
            sds[b][1].wait()
        return 0
    lax.fori_loop(0, NB_W // GRP, jb, 0)
    plsc.subcore_barrier()
    pltpu.sync_copy(den_sh.at[pl.ds(sid * STRIPE, STRIPE)],
                    den_hbm.at[cid].at[pl.ds(sid * STRIPE, STRIPE)])


_edge_w = pl.kernel(
    _edge_w_body,
    out_type=(
        jax.ShapeDtypeStruct((NW, NB_W, KB, L), jnp.float32),
        jax.ShapeDtypeStruct((NC, NPAD, L), jnp.float32),
    ),
    mesh=_MESH,
    compiler_params=_SC_PARAMS,
    scratch_types=[
        pltpu.VMEM((NB_W, KB), jnp.int32),
        pltpu.VMEM((NB_W, KB), jnp.int32),
        pltpu.VMEM((GRP, KB, L), jnp.float32),
        pltpu.VMEM((GRP, KB, L), jnp.float32),
        pltpu.VMEM((GRP, KB, L), jnp.float32),
        pltpu.VMEM_SHARED((NPAD, L), jnp.float32),
        pltpu.SemaphoreType.DMA,
        pltpu.SemaphoreType.DMA,
        pltpu.SemaphoreType.DMA,
        pltpu.SemaphoreType.DMA,
    ],
)


NH = 2

def _make_msg(nch, per_head, s0):

    def body(xw_st, srcs_hbm, dsts_hbm, w_hbm, out_hbm,
             idx_s, idx_d, rbuf, wbuf, zbuf, acc_sh, sem_g, sem_h, sem_s):
        cid = lax.axis_index("c")
        sid = lax.axis_index("s")

        def zr(r, _):
            for t in range(8):
                zbuf[r, pl.ds(t * L, L)] = jnp.zeros((L,), jnp.float32)
            return 0
        lax.fori_loop(0, KB, zr, 0)

        clo = jnp.where(cid == 0, 0, s0)
        chi = jnp.where(cid == 0, s0, nch)

        def chunk_body(ci, _):
            if per_head:
                hlv = jnp.full((L,), ci // 2, jnp.int32)
            else:
                hlv = jnp.zeros((L,), jnp.int32)
            for t in range(STRIPE // KB):
                pltpu.sync_copy(zbuf, acc_sh.at[pl.ds(sid * STRIPE + t * KB, KB)])
            plsc.subcore_barrier()

            for half in range(NH):
                pltpu.sync_copy(srcs_hbm.at[sid].at[half], idx_s)
                pltpu.sync_copy(dsts_hbm.at[sid].at[half], idx_d)

                def jb(g, _):
                    gds = []
                    for b in range(GRP):
                        j = g * GRP + b
                        gds.append((
                            pltpu.async_copy(xw_st.at[ci].at[idx_s.at[j]],
                                             rbuf.at[b], sem_g),
                            pltpu.async_copy(w_hbm.at[sid].at[half].at[j],
                                             wbuf.at[b], sem_h)))
                    sds = []
                    for b in range(GRP):
                        j = g * GRP + b
                        gds[b][0].wait()
                        gds[b][1].wait()

                        def kb8(kk, _):
                            for e in range(8):
                                k = kk * 8 + e
                                wv = plsc.load_gather(
                                    wbuf.at[b],
                                    [jnp.full((L,), k, jnp.int32), hlv])
                                for t in range(8):
                                    rbuf[b, k, pl.ds(t * L, L)] = (
                                        rbuf[b, k, pl.ds(t * L, L)] * wv)
                            return 0
                        lax.fori_loop(0, KB // 8, kb8, 0)
                        sds.append(pltpu.async_copy(
                            rbuf.at[b], acc_sh.at[idx_d.at[j]], sem_s,
                            add=True))
                    for b in range(GRP):
                        sds[b].wait()
                    return 0
                lax.fori_loop(0, NB_W // GRP, jb, 0)
            plsc.subcore_barrier()
            pltpu.sync_copy(
                acc_sh.at[pl.ds(sid * STRIPE, STRIPE)],
                out_hbm.at[pl.ds(sid * STRIPE, STRIPE),
                           pl.ds(ci * 128, 128)])
            plsc.subcore_barrier()
            return 0
        lax.fori_loop(clo, chi, chunk_body, 0)

    return pl.kernel(
        body,
        out_type=jax.ShapeDtypeStruct((NPAD, nch * 128), jnp.float32),
        mesh=_MESH,
        compiler_params=_SC_PARAMS,
        scratch_types=[
            pltpu.VMEM((NB_W, KB), jnp.int32),
            pltpu.VMEM((NB_W, KB), jnp.int32),
            pltpu.VMEM((GRP, KB, 128), jnp.float32),
            pltpu.VMEM((GRP, KB, L), jnp.float32),
            pltpu.VMEM((KB, 128), jnp.float32),
            pltpu.VMEM_SHARED((NPAD, 128), jnp.float32),
            pltpu.SemaphoreType.DMA,
            pltpu.SemaphoreType.DMA,
            pltpu.SemaphoreType.DMA,
        ],
    )


_msg16 = _make_msg(16, True, 10)
_msg2 = _make_msg(2, False, 1)


def _mm1_body(x_ref, w_ref, asd_ref, xw_ref, la_ref):
    c = pl.program_id(1)
    acc = jnp.dot(x_ref[...], w_ref[...], preferred_element_type=jnp.float32)
    xw_ref[...] = acc.reshape(1, 256, 128)
    la = jnp.dot(acc, asd_ref[...], preferred_element_type=jnp.float32)

    @pl.when(c == 0)
    def _():
        la_ref[...] = la

    @pl.when(c != 0)
    def _():
        la_ref[...] = la_ref[...] + la


def _mm1(x, w1, asd1):
    return pl.pallas_call(
        _mm1_body,
        grid=(NPAD // 256, 16),
        in_specs=[
            pl.BlockSpec((256, D), lambda i, c: (i, 0)),
            pl.BlockSpec((D, 128), lambda i, c: (0, c)),
            pl.BlockSpec((128, 128), lambda i, c: (c, 0)),
        ],
        out_specs=[
            pl.BlockSpec((1, 256, 128), lambda i, c: (c, i, 0)),
            pl.BlockSpec((256, 128), lambda i, c: (i, 0)),
        ],
        out_shape=[
            jax.ShapeDtypeStruct((16, NPAD, 128), jnp.float32),
            jax.ShapeDtypeStruct((NPAD, 128), jnp.float32),
        ],
    )(x, w1, asd1)


def _mm2_body(h_ref, w_ref, asd_ref, xw_ref, la_ref):
    c = pl.program_id(1)
    acc = jnp.dot(h_ref[...], w_ref[...], preferred_element_type=jnp.float32)
    xw_ref[...] = acc.reshape(1, 256, 128)
    la = jnp.dot(acc, asd_ref[...], preferred_element_type=jnp.float32)

    @pl.when(c == 0)
    def _():
        la_ref[...] = la

    @pl.when(c != 0)
    def _():
        la_ref[...] = la_ref[...] + la


def _mm2(h, w2, asd2):
    return pl.pallas_call(
        _mm2_body,
        grid=(NPAD // 256, 2),
        in_specs=[
            pl.BlockSpec((256, HEADS * HID), lambda i, c: (i, 0)),
            pl.BlockSpec((HEADS * HID, 128), lambda i, c: (0, c)),
            pl.BlockSpec((128, 128), lambda i, c: (c, 0)),
        ],
        out_specs=[
            pl.BlockSpec((1, 256, 128), lambda i, c: (c, i, 0)),
            pl.BlockSpec((256, 128), lambda i, c: (i, 0)),
        ],
        out_shape=[
            jax.ShapeDtypeStruct((2, NPAD, 128), jnp.float32),
            jax.ShapeDtypeStruct((NPAD, 128), jnp.float32),
        ],
    )(h, w2, asd2)


def _epi_body(num_ref, den_ref, m_ref, b_ref, g_ref, be_ref, out_ref):
    den = den_ref[0] + den_ref[1]
    dexp = jnp.dot(den, m_ref[...], preferred_element_type=jnp.float32)
    h = num_ref[...] / (dexp + 1e-16) + b_ref[...]
    mu = jnp.mean(h, axis=-1, keepdims=True)
    var = jnp.mean((h - mu) ** 2, axis=-1, keepdims=True)
    hn = (h - mu) / jnp.sqrt(var + 1e-5) * g_ref[...] + be_ref[...]
    out_ref[...] = jnp.where(hn > 0.0, hn, jnp.exp(hn) - 1.0)


def _make_epi(dt):
    def run(num, den, m, b, g, be):
        return pl.pallas_call(
            _epi_body,
            grid=(NPAD // 256,),
            in_specs=[
                pl.BlockSpec((256, dt), lambda i: (i, 0)),
                pl.BlockSpec((2, 256, L), lambda i: (0, i, 0)),
                pl.BlockSpec((L, dt), lambda i: (0, 0)),
                pl.BlockSpec((1, dt), lambda i: (0, 0)),
                pl.BlockSpec((1, dt), lambda i: (0, 0)),
                pl.BlockSpec((1, dt), lambda i: (0, 0)),
            ],
            out_specs=pl.BlockSpec((256, dt), lambda i: (i, 0)),
            out_shape=jax.ShapeDtypeStruct((NPAD, dt), jnp.float32),
        )(num, den, m, b, g, be)
    return run


_epi1 = _make_epi(HEADS * HID)
_epi2 = _make_epi(D)


def kernel(entity_ids, edge_index, entity_table,
           W1, a_src1, a_dst1, b1, g1, be1,
           W2, a_src2, a_dst2, b2, g2, be2):
    f32 = jnp.float32
    ids = entity_ids.astype(jnp.int32)
    pad_ids = jnp.concatenate(
        [ids, jnp.zeros((NPAD - N_NODES,), jnp.int32)]).reshape(
            NW, ROWS_W // KB, KB)
    loop = jnp.arange(N_NODES, dtype=jnp.int32)
    padc = jnp.full((EPAD - N_EDGES - N_NODES,), NPAD - 1, jnp.int32)
    src = jnp.concatenate([edge_index[0].astype(jnp.int32), loop, padc])
    dst = jnp.concatenate([edge_index[1].astype(jnp.int32), loop, padc])
    srcw = src.reshape(NW, NB_W, KB)
    dstw = dst.reshape(NW, NB_W, KB)
    srcs = src.reshape(NS, NH, NB_W, KB)
    dsts = dst.reshape(NS, NH, NB_W, KB)

    eye8 = jnp.eye(HEADS, dtype=f32)
    as_mat1 = (eye8[:, None, :] * a_src1[:, :, None]).reshape(HEADS * HID, HEADS)
    ad_mat1 = (eye8[:, None, :] * a_dst1[:, :, None]).reshape(HEADS * HID, HEADS)
    z8 = jnp.zeros((HEADS * HID, 8), f32)
    asd1 = jnp.concatenate(
        [as_mat1, z8, ad_mat1, jnp.zeros((HEADS * HID, 128 - 24), f32)], axis=1)
    asd2 = jnp.concatenate(
        [a_src2.T, jnp.zeros((D, 15), f32),
         a_dst2.T, jnp.zeros((D, 128 - 17), f32)], axis=1)
    m1 = jnp.concatenate(
        [jnp.kron(eye8, jnp.ones((1, HID), f32)),
         jnp.zeros((8, HEADS * HID), f32)], axis=0)
    m2 = jnp.concatenate([jnp.ones((1, D), f32), jnp.zeros((15, D), f32)], axis=0)

    x = _gather_rows(entity_table, pad_ids)
    xw_ch, asad1 = _mm1(x, W1, asd1)
    w1e, den1 = _edge_w(asad1[:, 0:16], asad1[:, 16:32], srcw, dstw)
    num1 = _msg16(xw_ch, srcs, dsts, w1e.reshape(NS, NH, NB_W, KB, L))
    h1 = _epi1(num1, den1, m1, b1.reshape(1, -1), g1.reshape(1, -1),
               be1.reshape(1, -1))
    xw2_ch, asad2_l = _mm2(h1, W2, asd2)
    w2e, den2 = _edge_w(asad2_l[:, 0:16], asad2_l[:, 16:32], srcw, dstw)
    num2 = _msg2(xw2_ch, srcs, dsts, w2e.reshape(NS, NH, NB_W, KB, L))
    h2 = _epi2(num2, den2, m2, b2.reshape(1, -1), g2.reshape(1, -1),
               be2.reshape(1, -1))
    return h2[:N_NODES]
